# baseline (speedup 1.0000x reference)
import functools

import jax
import jax.numpy as jnp
from jax import lax
from jax.experimental import pallas as pl
from jax.experimental.pallas import tpu as pltpu
from jax.experimental.pallas import tpu_sc as plsc

EMBED = 64
PLOT_DIM = 384
HIDDEN = 128
IDX_CHUNK = 128


def _sc_counts():
    try:
        info = plsc.get_sparse_core_info()
        return int(info.num_cores), int(info.num_subcores)
    except Exception:
        return 2, 16


def _make_gather(batch):
    NC, NS = _sc_counts()
    NW = NC * NS
    b_per_w = batch // NW
    n_chunks = b_per_w // IDX_CHUNK
    assert b_per_w % IDX_CHUNK == 0
    rows_per_w = b_per_w // IDX_CHUNK

    mesh = plsc.VectorSubcoreMesh(core_axis_name="c", subcore_axis_name="s")

    @functools.partial(
        pl.kernel,
        out_type=[
            jax.ShapeDtypeStruct((batch, 2 * EMBED), jnp.float32),
            jax.ShapeDtypeStruct((batch, 2 * EMBED), jnp.float32),
        ],
        mesh=mesh,
        scratch_types=[
            pltpu.VMEM((rows_per_w, IDX_CHUNK), jnp.int32),
            pltpu.VMEM((rows_per_w, IDX_CHUNK), jnp.int32),
            pltpu.VMEM((b_per_w, 2 * EMBED), jnp.float32),
            pltpu.SemaphoreType.DMA,
        ],
    )
    def gather2(users_hbm, movies_hbm, ut_hbm, mt_hbm, ue_out, me_out,
                uidx_v, midx_v, rows_v, sem):
        wid = lax.axis_index("s") * NC + lax.axis_index("c")
        base = wid * b_per_w
        row0 = wid * rows_per_w
        pltpu.sync_copy(users_hbm.at[pl.ds(row0, rows_per_w)], uidx_v)
        pltpu.sync_copy(movies_hbm.at[pl.ds(row0, rows_per_w)], midx_v)
        copies = []
        for j in range(n_chunks):
            dst = rows_v.at[pl.ds(j * IDX_CHUNK, IDX_CHUNK)]
            copies.append(pltpu.async_copy(ut_hbm.at[uidx_v.at[j]], dst, sem))
        for c in copies:
            c.wait()
        pltpu.sync_copy(rows_v, ue_out.at[pl.ds(base, b_per_w)])
        copies = []
        for j in range(n_chunks):
            dst = rows_v.at[pl.ds(j * IDX_CHUNK, IDX_CHUNK)]
            copies.append(pltpu.async_copy(mt_hbm.at[midx_v.at[j]], dst, sem))
        for c in copies:
            c.wait()
        pltpu.sync_copy(rows_v, me_out.at[pl.ds(base, b_per_w)])

    return gather2


def _mlp_body(ue_ref, me_ref, up_ref, mp_ref, plot_ref, w1_ref, b1_ref,
              w2r_ref, b2_ref, out_ref):
    up = up_ref[...]
    mp = mp_ref[...]
    ue = ue_ref[:, 0:EMBED] * (1.0 - up) + ue_ref[:, EMBED:2 * EMBED] * up
    me = me_ref[:, 0:EMBED] * (1.0 - mp) + me_ref[:, EMBED:2 * EMBED] * mp
    x = jnp.dot(ue, w1_ref[0:EMBED, :], preferred_element_type=jnp.float32)
    x += jnp.dot(me, w1_ref[EMBED:2 * EMBED, :],
                 preferred_element_type=jnp.float32)
    x += jnp.dot(plot_ref[...], w1_ref[2 * EMBED:, :],
                 preferred_element_type=jnp.float32)
    x = jnp.maximum(x + b1_ref[...], 0.0)
    out_ref[...] = (jnp.sum(x * w2r_ref[...], axis=1, keepdims=True)
                    + b2_ref[...])


def _make_mlp(batch, blk):
    grid = batch // blk
    in_dim = 2 * EMBED + PLOT_DIM
    return pl.pallas_call(
        _mlp_body,
        grid=(grid,),
        in_specs=[
            pl.BlockSpec((blk, 2 * EMBED), lambda i: (i, 0)),
            pl.BlockSpec((blk, 2 * EMBED), lambda i: (i, 0)),
            pl.BlockSpec((blk, 1), lambda i: (i, 0)),
            pl.BlockSpec((blk, 1), lambda i: (i, 0)),
            pl.BlockSpec((blk, PLOT_DIM), lambda i: (i, 0)),
            pl.BlockSpec((in_dim, HIDDEN), lambda i: (0, 0)),
            pl.BlockSpec((1, HIDDEN), lambda i: (0, 0)),
            pl.BlockSpec((1, HIDDEN), lambda i: (0, 0)),
            pl.BlockSpec((1, 1), lambda i: (0, 0)),
        ],
        out_specs=pl.BlockSpec((blk, 1), lambda i: (i, 0)),
        out_shape=jax.ShapeDtypeStruct((batch, 1), jnp.float32),
    )


@jax.jit
def kernel(users, movies, plot_embeddings, user_table, movie_table,
           W1, b1, W2, b2):
    batch = users.shape[0]
    users = users.astype(jnp.int32)
    movies = movies.astype(jnp.int32)
    upair = (users >> 1).reshape(-1, IDX_CHUNK)
    mpair = (movies >> 1).reshape(-1, IDX_CHUNK)
    uparity = (users & 1).astype(jnp.float32).reshape(batch, 1)
    mparity = (movies & 1).astype(jnp.float32).reshape(batch, 1)
    ut2 = user_table.reshape(-1, 2 * EMBED)
    mt2 = movie_table.reshape(-1, 2 * EMBED)
    ue, me = _make_gather(batch)(upair, mpair, ut2, mt2)
    mlp = _make_mlp(batch, 2048)
    return mlp(ue, me, uparity, mparity, plot_embeddings,
               W1, b1.reshape(1, HIDDEN), W2.reshape(1, HIDDEN),
               b2.reshape(1, 1))
